# Initial kernel scaffold; baseline (speedup 1.0000x reference)
import dataclasses
import math

import jax
import jax.numpy as jnp
import numpy as np
from jax import lax
from jax.experimental import pallas as pl
from jax.experimental.pallas import tpu as pltpu
from jax.experimental.pallas import tpu_sc as plsc

D = 64
SEQ = 200
B = 1024
N = SEQ * B
NW = 32
PER_W = N // NW
CH = 128
NCH = PER_W // CH
SCALE = 8.0

HOUR_OFF = 0
MIN_OFF = 24
WD_OFF = 28
MODE_OFF = 35
PE_OFF = 43
SV_ROWS = PE_OFF + SEQ


def _pos_encoding_np():
    den = np.exp(-np.arange(0, D, 2, dtype=np.float32) * (math.log(10000.0) / D))
    pos = np.arange(0, SEQ, dtype=np.float32).reshape(SEQ, 1)
    pe = np.zeros((SEQ, D), dtype=np.float32)
    pe[:, 0::2] = np.sin(pos * den)
    pe[:, 1::2] = np.cos(pos * den)
    return pe


_PE = _pos_encoding_np()
_S_IDX = (PE_OFF + (np.arange(N, dtype=np.int64) // B)).astype(np.int32)


def _sc_kernel_body(idx_hbm, smalls_hbm, loc_hbm, out_hbm,
                    ib0, ib1, rows0, rows1, sv, tt, twm,
                    gsem0, gsem1, osem0, osem1):
    ib = (ib0, ib1)
    rows = (rows0, rows1)
    gsem = (gsem0, gsem1)
    osem = (osem0, osem1)

    wid = lax.axis_index("subcore") * 2 + lax.axis_index("core")
    start = wid * PER_W

    pltpu.sync_copy(smalls_hbm, sv)

    @pl.loop(0, 96)
    def _(t):
        h = t // 4
        m = t % 4
        for j in range(4):
            tt[pl.ds(t * D + j * 16, 16)] = (
                sv[pl.ds((HOUR_OFF + h) * D + j * 16, 16)]
                + sv[pl.ds((MIN_OFF + m) * D + j * 16, 16)]
            ) * SCALE

    @pl.loop(0, 56)
    def _(i):
        wd = i // 8
        mo = i % 8
        for j in range(4):
            twm[pl.ds(i * D + j * 16, 16)] = (
                sv[pl.ds((WD_OFF + wd) * D + j * 16, 16)]
                + sv[pl.ds((MODE_OFF + mo) * D + j * 16, 16)]
            ) * SCALE

    def fire_gather(c, bi):
        base = start + c * CH
        pltpu.sync_copy(idx_hbm.at[:, pl.ds(base, CH)], ib[bi])
        pltpu.make_async_copy(loc_hbm.at[ib[bi].at[0]], rows[bi], gsem[bi]).start()

    def wait_gather(bi):
        pltpu.make_async_copy(loc_hbm.at[ib[bi].at[0]], rows[bi], gsem[bi]).wait()

    def fire_out(c, bi):
        base = start + c * CH
        pltpu.make_async_copy(rows[bi], out_hbm.at[pl.ds(base, CH)], osem[bi]).start()

    def wait_out(bi):
        pltpu.make_async_copy(rows[bi], out_hbm.at[pl.ds(start, CH)], osem[bi]).wait()

    def compute_chunk(ibuf, rbuf):
        @pl.loop(0, CH // 16)
        def _(g):
            gsl = pl.ds(g * 16, 16)
            t64 = ibuf[1, gsl] * D
            wm64 = (ibuf[2, gsl] * 8 + ibuf[3, gsl]) * D
            s64 = ibuf[4, gsl] * D
            e16 = lax.broadcasted_iota(jnp.int32, (16,), 0) + g * 16

            @pl.loop(0, D, step=4)
            def _(k0):
                for dk in range(4):
                    k = k0 + dk
                    kb = jnp.zeros((16,), jnp.int32) + k
                    loc = plsc.load_gather(rbuf, [e16, kb])
                    a_tt = plsc.load_gather(tt, [t64 + kb])
                    a_tw = plsc.load_gather(twm, [wm64 + kb])
                    a_pe = plsc.load_gather(sv, [s64 + kb])
                    v = loc * SCALE + a_tt + a_tw + a_pe
                    plsc.store_scatter(rbuf, [e16, kb], v)

    fire_gather(0, 0)

    @pl.loop(0, NCH // 2)
    def _(i):
        for b01 in (0, 1):
            c = i * 2 + b01
            nb = 1 - b01
            wait_gather(b01)

            @pl.when(c + 1 < NCH)
            def _():
                @pl.when(c >= 1)
                def _():
                    wait_out(nb)

                fire_gather(c + 1, nb)

            compute_chunk(ib[b01], rows[b01])
            fire_out(c, b01)

    wait_out(0)
    wait_out(1)


def kernel(src, time, weekday, mode, emb_loc_w, emb_mode_w, minute_w, hour_w, weekday_w):
    idx_packed = jnp.stack([
        src.reshape(-1).astype(jnp.int32),
        time.reshape(-1).astype(jnp.int32),
        weekday.reshape(-1).astype(jnp.int32),
        mode.reshape(-1).astype(jnp.int32),
        jnp.asarray(_S_IDX),
    ])
    smalls = jnp.concatenate(
        [hour_w, minute_w, weekday_w, emb_mode_w, jnp.asarray(_PE)], axis=0
    ).reshape(-1)

    mesh = plsc.VectorSubcoreMesh(core_axis_name="core", subcore_axis_name="subcore")

    cp = pltpu.CompilerParams()
    if "needs_layout_passes" in pltpu.CompilerParams.__dataclass_fields__:
        cp = dataclasses.replace(cp, needs_layout_passes=False)

    run = pl.kernel(
        _sc_kernel_body,
        out_type=jax.ShapeDtypeStruct((N, D), jnp.float32),
        mesh=mesh,
        compiler_params=cp,
        scratch_types=[
            pltpu.VMEM((5, CH), jnp.int32),
            pltpu.VMEM((5, CH), jnp.int32),
            pltpu.VMEM((CH, D), jnp.float32),
            pltpu.VMEM((CH, D), jnp.float32),
            pltpu.VMEM((SV_ROWS * D,), jnp.float32),
            pltpu.VMEM((96 * D,), jnp.float32),
            pltpu.VMEM((56 * D,), jnp.float32),
            pltpu.SemaphoreType.DMA,
            pltpu.SemaphoreType.DMA,
            pltpu.SemaphoreType.DMA,
            pltpu.SemaphoreType.DMA,
        ],
    )
    out = run(idx_packed, smalls, emb_loc_w)
    return out.reshape(SEQ, B, D)


# speedup vs baseline: 1.7598x; 1.7598x over previous
import dataclasses
import math

import jax
import jax.numpy as jnp
import numpy as np
from jax import lax
from jax.experimental import pallas as pl
from jax.experimental.pallas import tpu as pltpu
from jax.experimental.pallas import tpu_sc as plsc

D = 64
SEQ = 200
B = 1024
N = SEQ * B
NW = 32
PER_W = N // NW
CH = 128
NCH = PER_W // CH
SCALE = 8.0

HOUR_OFF = 0
MIN_OFF = 24
WD_OFF = 28
MODE_OFF = 35
PE_OFF = 43
SV_ROWS = PE_OFF + SEQ


def _pos_encoding_np():
    den = np.exp(-np.arange(0, D, 2, dtype=np.float32) * (math.log(10000.0) / D))
    pos = np.arange(0, SEQ, dtype=np.float32).reshape(SEQ, 1)
    pe = np.zeros((SEQ, D), dtype=np.float32)
    pe[:, 0::2] = np.sin(pos * den)
    pe[:, 1::2] = np.cos(pos * den)
    return pe


_PE = _pos_encoding_np()
_S_IDX = (PE_OFF + (np.arange(N, dtype=np.int64) // B)).astype(np.int32)


def _sc_kernel_body(idx_hbm, smalls_hbm, loc_hbm, out_hbm,
                    ib0, ib1, rows0, rows1, sv, tt, twm,
                    gsem0, gsem1, osem0, osem1):
    ib = (ib0, ib1)
    rows = (rows0, rows1)
    gsem = (gsem0, gsem1)
    osem = (osem0, osem1)

    wid = lax.axis_index("subcore") * 2 + lax.axis_index("core")
    start = wid * PER_W

    pltpu.sync_copy(smalls_hbm, sv)

    @pl.loop(0, 96)
    def _(t):
        h = t // 4
        m = t % 4
        for j in range(4):
            tt[pl.ds(t * D + j * 16, 16)] = (
                sv[pl.ds((HOUR_OFF + h) * D + j * 16, 16)]
                + sv[pl.ds((MIN_OFF + m) * D + j * 16, 16)]
            ) * SCALE

    @pl.loop(0, 56)
    def _(i):
        wd = i // 8
        mo = i % 8
        for j in range(4):
            twm[pl.ds(i * D + j * 16, 16)] = (
                sv[pl.ds((WD_OFF + wd) * D + j * 16, 16)]
                + sv[pl.ds((MODE_OFF + mo) * D + j * 16, 16)]
            ) * SCALE

    def fire_gather(c, bi):
        base = start + c * CH
        pltpu.sync_copy(idx_hbm.at[:, pl.ds(base, CH)], ib[bi])
        pltpu.make_async_copy(loc_hbm.at[ib[bi].at[0]], rows[bi], gsem[bi]).start()

    def wait_gather(bi):
        pltpu.make_async_copy(loc_hbm.at[ib[bi].at[0]], rows[bi], gsem[bi]).wait()

    def fire_out(c, bi):
        base = start + c * CH
        pltpu.make_async_copy(rows[bi], out_hbm.at[pl.ds(base, CH)], osem[bi]).start()

    def wait_out(bi):
        pltpu.make_async_copy(rows[bi], out_hbm.at[pl.ds(start, CH)], osem[bi]).wait()

    def compute_chunk(ibuf, rbuf):
        @pl.loop(0, CH // 16)
        def _(g):
            gsl = pl.ds(g * 16, 16)
            t64 = ibuf[1, gsl] * D
            wm64 = (ibuf[2, gsl] * 8 + ibuf[3, gsl]) * D
            s64 = ibuf[4, gsl] * D
            e16 = lax.broadcasted_iota(jnp.int32, (16,), 0) + g * 16

            @pl.loop(0, D, step=4)
            def _(k0):
                for dk in range(4):
                    k = k0 + dk
                    kb = jnp.zeros((16,), jnp.int32) + k
                    loc = plsc.load_gather(rbuf, [e16, kb])
                    a_tt = plsc.load_gather(tt, [t64 + kb])
                    a_tw = plsc.load_gather(twm, [wm64 + kb])
                    a_pe = plsc.load_gather(sv, [s64 + kb])
                    v = loc * SCALE + a_tt + a_tw + a_pe
                    plsc.store_scatter(rbuf, [e16, kb], v)

    fire_gather(0, 0)

    @pl.loop(0, NCH // 2)
    def _(i):
        for b01 in (0, 1):
            c = i * 2 + b01
            nb = 1 - b01
            wait_gather(b01)

            @pl.when(c + 1 < NCH)
            def _():
                @pl.when(c >= 1)
                def _():
                    wait_out(nb)

                fire_gather(c + 1, nb)

            compute_chunk(ib[b01], rows[b01])
            fire_out(c, b01)

    wait_out(0)
    wait_out(1)


def kernel(src, time, weekday, mode, emb_loc_w, emb_mode_w, minute_w, hour_w, weekday_w):
    idx_packed = jnp.stack([
        src.reshape(-1).astype(jnp.int32),
        time.reshape(-1).astype(jnp.int32),
        weekday.reshape(-1).astype(jnp.int32),
        mode.reshape(-1).astype(jnp.int32),
        jnp.asarray(_S_IDX),
    ])
    smalls = jnp.concatenate(
        [hour_w, minute_w, weekday_w, emb_mode_w, jnp.asarray(_PE)], axis=0
    ).reshape(-1)

    mesh = plsc.VectorSubcoreMesh(core_axis_name="core", subcore_axis_name="subcore")

    cp = pltpu.CompilerParams(use_tc_tiling_on_sc=False)
    if "needs_layout_passes" in pltpu.CompilerParams.__dataclass_fields__:
        cp = dataclasses.replace(cp, needs_layout_passes=False)

    run = pl.kernel(
        _sc_kernel_body,
        out_type=jax.ShapeDtypeStruct((N, D), jnp.float32),
        mesh=mesh,
        compiler_params=cp,
        scratch_types=[
            pltpu.VMEM((5, CH), jnp.int32),
            pltpu.VMEM((5, CH), jnp.int32),
            pltpu.VMEM((CH, D), jnp.float32),
            pltpu.VMEM((CH, D), jnp.float32),
            pltpu.VMEM((SV_ROWS * D,), jnp.float32),
            pltpu.VMEM((96 * D,), jnp.float32),
            pltpu.VMEM((56 * D,), jnp.float32),
            pltpu.SemaphoreType.DMA,
            pltpu.SemaphoreType.DMA,
            pltpu.SemaphoreType.DMA,
            pltpu.SemaphoreType.DMA,
        ],
    )
    out = run(idx_packed, smalls, emb_loc_w)
    return out.reshape(SEQ, B, D)


# speedup vs baseline: 3.7584x; 2.1356x over previous
import dataclasses
import math

import jax
import jax.numpy as jnp
import numpy as np
from jax import lax
from jax.experimental import pallas as pl
from jax.experimental.pallas import tpu as pltpu
from jax.experimental.pallas import tpu_sc as plsc

D = 64
SEQ = 200
B = 1024
N = SEQ * B
NW = 32
PER_W = N // NW
CH = 128
NCH = PER_W // CH
SCALE = 8.0

HOUR_OFF = 0
MIN_OFF = 24
WD_OFF = 28
MODE_OFF = 35
PE_OFF = 43
SV_ROWS = PE_OFF + SEQ


def _pos_encoding_np():
    den = np.exp(-np.arange(0, D, 2, dtype=np.float32) * (math.log(10000.0) / D))
    pos = np.arange(0, SEQ, dtype=np.float32).reshape(SEQ, 1)
    pe = np.zeros((SEQ, D), dtype=np.float32)
    pe[:, 0::2] = np.sin(pos * den)
    pe[:, 1::2] = np.cos(pos * den)
    return pe


_PE = _pos_encoding_np()


def _sc_kernel_body(idx_hbm, smalls_hbm, loc_hbm, out_hbm,
                    ib0, ib1, wm0, wm1, rows0, rows1, att0, att1, atw0, atw1,
                    sv, tt, twm,
                    gsem0, gsem1, tsem0, tsem1, wsem0, wsem1, osem0, osem1):
    ib = (ib0, ib1)
    wmb = (wm0, wm1)
    rows = (rows0, rows1)
    att = (att0, att1)
    atw = (atw0, atw1)
    gsem = (gsem0, gsem1)
    tsem = (tsem0, tsem1)
    wsem = (wsem0, wsem1)
    osem = (osem0, osem1)

    wid = lax.axis_index("subcore") * 2 + lax.axis_index("core")
    start = wid * PER_W

    pltpu.sync_copy(smalls_hbm, sv)

    @pl.when(lax.axis_index("subcore") == 0)
    def _():
        @pl.loop(0, 96)
        def _(t):
            h = t // 4
            m = t % 4
            for j in range(4):
                sl = pl.ds(j * 16, 16)
                att0[t, sl] = (sv[HOUR_OFF + h, sl] + sv[MIN_OFF + m, sl]) * SCALE

        @pl.loop(0, 56)
        def _(i):
            wd = i // 8
            mo = i % 8
            for j in range(4):
                sl = pl.ds(j * 16, 16)
                atw0[i, sl] = (sv[WD_OFF + wd, sl] + sv[MODE_OFF + mo, sl]) * SCALE

        pltpu.sync_copy(att0.at[pl.ds(0, 96)], tt)
        pltpu.sync_copy(atw0.at[pl.ds(0, 56)], twm)

    plsc.subcore_barrier()

    def fire_gathers(c, bi):
        base = start + c * CH
        pltpu.sync_copy(idx_hbm.at[:, pl.ds(base, CH)], ib[bi])

        @pl.loop(0, CH // 16)
        def _(g):
            gsl = pl.ds(g * 16, 16)
            wmb[bi][gsl] = ib[bi][2, gsl] * 8 + ib[bi][3, gsl]

        pltpu.make_async_copy(loc_hbm.at[ib[bi].at[0]], rows[bi], gsem[bi]).start()
        pltpu.make_async_copy(tt.at[ib[bi].at[1]], att[bi], tsem[bi]).start()
        pltpu.make_async_copy(twm.at[wmb[bi]], atw[bi], wsem[bi]).start()

    def wait_gathers(bi):
        pltpu.make_async_copy(loc_hbm.at[ib[bi].at[0]], rows[bi], gsem[bi]).wait()
        pltpu.make_async_copy(tt.at[ib[bi].at[1]], att[bi], tsem[bi]).wait()
        pltpu.make_async_copy(twm.at[wmb[bi]], atw[bi], wsem[bi]).wait()

    def fire_out(c, bi):
        base = start + c * CH
        pltpu.make_async_copy(att[bi], out_hbm.at[pl.ds(base, CH)], osem[bi]).start()

    def wait_out(bi):
        pltpu.make_async_copy(att[bi], out_hbm.at[pl.ds(start, CH)], osem[bi]).wait()

    def compute_chunk(c, bi):
        s = (start + c * CH) // B
        pe_regs = [sv[PE_OFF + s, pl.ds(j * 16, 16)] for j in range(4)]
        rb = rows[bi]
        ab = att[bi]
        wb = atw[bi]

        @pl.loop(0, CH, step=2)
        def _(e0):
            for de in range(2):
                e = e0 + de
                for j in range(4):
                    sl = pl.ds(j * 16, 16)
                    ab[e, sl] = rb[e, sl] * SCALE + ab[e, sl] + wb[e, sl] + pe_regs[j]

    fire_gathers(0, 0)

    @pl.loop(0, NCH // 2)
    def _(i):
        for b01 in (0, 1):
            c = i * 2 + b01
            nb = 1 - b01
            wait_gathers(b01)

            @pl.when(c + 1 < NCH)
            def _():
                @pl.when(c >= 1)
                def _():
                    wait_out(nb)

                fire_gathers(c + 1, nb)

            compute_chunk(c, b01)
            fire_out(c, b01)

    wait_out(0)
    wait_out(1)


def kernel(src, time, weekday, mode, emb_loc_w, emb_mode_w, minute_w, hour_w, weekday_w):
    idx_packed = jnp.stack([
        src.reshape(-1).astype(jnp.int32),
        time.reshape(-1).astype(jnp.int32),
        weekday.reshape(-1).astype(jnp.int32),
        mode.reshape(-1).astype(jnp.int32),
    ])
    smalls = jnp.concatenate(
        [hour_w, minute_w, weekday_w, emb_mode_w, jnp.asarray(_PE)], axis=0
    )

    mesh = plsc.VectorSubcoreMesh(core_axis_name="core", subcore_axis_name="subcore")

    cp = pltpu.CompilerParams(use_tc_tiling_on_sc=False)
    if "needs_layout_passes" in pltpu.CompilerParams.__dataclass_fields__:
        cp = dataclasses.replace(cp, needs_layout_passes=False)

    run = pl.kernel(
        _sc_kernel_body,
        out_type=jax.ShapeDtypeStruct((N, D), jnp.float32),
        mesh=mesh,
        compiler_params=cp,
        scratch_types=[
            pltpu.VMEM((4, CH), jnp.int32),
            pltpu.VMEM((4, CH), jnp.int32),
            pltpu.VMEM((CH,), jnp.int32),
            pltpu.VMEM((CH,), jnp.int32),
            pltpu.VMEM((CH, D), jnp.float32),
            pltpu.VMEM((CH, D), jnp.float32),
            pltpu.VMEM((CH, D), jnp.float32),
            pltpu.VMEM((CH, D), jnp.float32),
            pltpu.VMEM((CH, D), jnp.float32),
            pltpu.VMEM((CH, D), jnp.float32),
            pltpu.VMEM((SV_ROWS, D), jnp.float32),
            pltpu.VMEM_SHARED((96, D), jnp.float32),
            pltpu.VMEM_SHARED((56, D), jnp.float32),
        ] + [pltpu.SemaphoreType.DMA] * 8,
    )
    out = run(idx_packed, smalls, emb_loc_w)
    return out.reshape(SEQ, B, D)
